# baseline (speedup 1.0000x reference)
import functools

import jax
import jax.numpy as jnp
from jax import lax
from jax.experimental import pallas as pl
from jax.experimental.pallas import tpu as pltpu
from jax.experimental.pallas import tpu_sc as plsc

NUM_SC = 2
NUM_TEC = 16
NUM_W = NUM_SC * NUM_TEC

CB = 128
CD = 64


def _dense_nodes_body(x_ref, wn_ref, bn_ref, wc_ref, bc_ref, t2_ref, p_ref):
    t = jnp.dot(x_ref[...], wn_ref[...], preferred_element_type=jnp.float32)
    t = t + bn_ref[...]
    p = jnp.dot(t, wc_ref[...], preferred_element_type=jnp.float32)
    p_ref[...] = p + bc_ref[...]
    half = t.shape[1] // 2
    t2_ref[0] = t[:, :half]
    t2_ref[1] = t[:, half:]


def _dense_edges_body(ef_ref, we_ref, be_ref, r_ref):
    r = jnp.dot(ef_ref[...], we_ref[...], preferred_element_type=jnp.float32)
    r_ref[...] = r + be_ref[...]


def _logits_body(n_ch, loop_n, p_hbm, r_hbm, src_hbm, h_hbm, part_hbm,
                 idx_all, rows0, rows1, r0, r1, h0, h1, psum_v, pack_v,
                 sg0, sg1, sh0, sh1):
    c = lax.axis_index("c")
    s = lax.axis_index("s")
    wid = s * NUM_SC + c
    lanes = lax.iota(jnp.int32, 16)

    base_row = jnp.minimum(wid * loop_n, n_ch - loop_n)
    base_flat = pl.multiple_of(base_row * CB, 8)
    pltpu.sync_copy(src_hbm.at[pl.ds(base_flat, loop_n * CB)], idx_all)
    for l in range(4):
        psum_v[pl.ds(16 * l, 16)] = jnp.zeros((16,), jnp.float32)

    rows_b = (rows0, rows1)
    r_b = (r0, r1)
    h_b = (h0, h1)
    sg_b = (sg0, sg1)
    sh_b = (sh0, sh1)

    def start(j, b):
        jj = jnp.minimum(j, loop_n - 1)
        ebase = pl.multiple_of((base_row + jj) * CB, 8)
        pltpu.async_copy(p_hbm.at[idx_all.at[pl.ds(jj * CB, CB)]],
                         rows_b[b], sg_b[b])
        pltpu.async_copy(r_hbm.at[pl.ds(ebase, CB)], r_b[b], sg_b[b])

    def process(j, b, first):
        jj = jnp.minimum(j, loop_n - 1)
        canon = jnp.where(base_row + jj >= wid * loop_n, 1.0, 0.0)
        pltpu.make_async_copy(p_hbm.at[idx_all.at[pl.ds(jj * CB, CB)]],
                              rows_b[b], sg_b[b]).wait()
        pltpu.make_async_copy(r_hbm.at[pl.ds(0, CB)], r_b[b],
                              sg_b[b]).wait()
        if not first:
            pltpu.make_async_copy(h_b[b], h_hbm.at[pl.ds(0, 4 * CB)],
                                  sh_b[b]).wait()
        rows_v = rows_b[b]
        r_v = r_b[b]
        h_v = h_b[b]

        def group(g, carry2):
            hvecs = [jnp.zeros((16,), jnp.float32) for _ in range(4)]
            for u in range(16):
                i = g * 16 + u
                rsegs = [r_v[i, pl.ds(16 * k, 16)] for k in range(4)]
                msk = lanes == u
                for l in range(4):
                    acc = rows_v[i, pl.ds(64 * l, 16)] * rsegs[0]
                    acc = acc + rows_v[i, pl.ds(64 * l + 16, 16)] * rsegs[1]
                    acc = acc + rows_v[i, pl.ds(64 * l + 32, 16)] * rsegs[2]
                    acc = acc + rows_v[i, pl.ds(64 * l + 48, 16)] * rsegs[3]
                    hvecs[l] = jnp.where(msk, plsc.cumsum(acc)[15], hvecs[l])
            for l in range(4):
                hv = hvecs[l]
                h_v[pl.ds(128 * l + 16 * g, 16)] = hv
                glv = jnp.where(hv >= 0, hv, 0.01 * hv)
                psum_v[pl.ds(16 * l, 16)] = (
                    psum_v[pl.ds(16 * l, 16)] + jnp.exp(glv) * canon)
            return carry2

        lax.fori_loop(0, CB // 16, group, 0, unroll=False)
        pltpu.async_copy(
            h_v, h_hbm.at[pl.ds((base_row + jj) * (4 * CB), 4 * CB)],
            sh_b[b])
        start(j + 2, b)

    start(0, 0)
    start(1, 1)
    process(0, 0, True)
    process(1, 1, True)

    def pair(g, carry):
        process(2 * g, 0, False)
        process(2 * g + 1, 1, False)
        return carry

    lax.fori_loop(1, (loop_n + 1) // 2, pair, 0, unroll=False)
    if loop_n % 2:
        process(loop_n - 1, 0, False)
    for b in range(2):
        pltpu.make_async_copy(p_hbm.at[idx_all.at[pl.ds(0, CB)]], rows_b[b],
                              sg_b[b]).wait()
        pltpu.make_async_copy(r_hbm.at[pl.ds(0, CB)], r_b[b],
                              sg_b[b]).wait()
        pltpu.make_async_copy(h_b[b], h_hbm.at[pl.ds(0, 4 * CB)],
                              sh_b[b]).wait()

    pack = jnp.zeros((16,), jnp.float32)
    for l in range(4):
        tot = plsc.cumsum(psum_v[pl.ds(16 * l, 16)])[15]
        pack = jnp.where(lanes == l, tot, pack)
    pack_v[...] = pack
    pltpu.sync_copy(pack_v, part_hbm.at[pl.ds(wid * 16, 16)])


def _scatter_body(n_nodes, n_ch, n_loop, out_ch, t2_hbm, h_hbm, part_hbm,
                  src_hbm, dst_hbm, out_hbm, a_hbm, idx0, idx1, dst0, dst1,
                  hb0, hb1, ab0, ab1, pbuf, rows0, rows1, srows0, srows1,
                  zbuf, acc_sh, sg0, sg1, ss0, ss1, si0, si1, sd0, sd1,
                  sa0, sa1, sao0, sao1, so):
    c = lax.axis_index("c")
    s = lax.axis_index("s")
    lanes = lax.iota(jnp.int32, 16)
    off = c * n_nodes

    pltpu.sync_copy(part_hbm, pbuf)
    zacc = jnp.zeros((16,), jnp.float32)
    for w in range(NUM_W):
        zacc = zacc + pbuf[pl.ds(16 * w, 16)]
    zinv = jnp.where(lanes < 4, 0.25 / zacc, 0.0)
    zs = [zinv[l] for l in range(4)]

    def zfill(r, carry):
        for k in range(8):
            zbuf[r, pl.ds(16 * k, 16)] = jnp.zeros((16,), jnp.float32)
        return carry

    lax.fori_loop(0, 16, zfill, 0, unroll=False)

    def rowbase(k):
        rb = jnp.minimum(s * (out_ch * 16) + 16 * k, n_nodes - 16)
        return pl.multiple_of(rb, 8)

    for k in range(out_ch + 1):
        pltpu.sync_copy(zbuf, acc_sh.at[pl.ds(rowbase(k), 16)])
    plsc.subcore_barrier()

    idx_b = (idx0, idx1)
    dst_b = (dst0, dst1)
    h_b = (hb0, hb1)
    a_b = (ab0, ab1)
    rows_b = (rows0, rows1)
    srows_b = (srows0, srows1)
    sg_b = (sg0, sg1)
    ss_b = (ss0, ss1)
    si_b = (si0, si1)
    sd_b = (sd0, sd1)
    sh_b = (sa0, sa1)
    sao_b = (sao0, sao1)

    def cid_of(j):
        return jnp.minimum(j * NUM_TEC + s, n_ch - 1)

    def ebase_of(j):
        return pl.multiple_of(cid_of(j) * CD, 8)

    def stage_h(j, b):
        cid = cid_of(j)
        bbase = (cid // 2) * (4 * CB) + (cid % 2) * CD
        for l in range(4):
            pltpu.async_copy(h_hbm.at[pl.ds(bbase + l * CB, CD)],
                             h_b[b].at[pl.ds(l * CD, CD)], sh_b[b])

    def wait_h(b):
        for l in range(4):
            pltpu.make_async_copy(h_hbm.at[pl.ds(0, CD)],
                                  h_b[b].at[pl.ds(l * CD, CD)],
                                  sh_b[b]).wait()

    def add_off(buf):
        for k in range(CD // 16):
            buf[pl.ds(16 * k, 16)] = buf[pl.ds(16 * k, 16)] + off

    def process(j, b, first):
        scale = jnp.where(j * NUM_TEC + s < n_ch, 1.0, 0.0)
        pltpu.make_async_copy(t2_hbm.at[idx_b[b]], rows_b[b], sg_b[b]).wait()
        pltpu.async_copy(src_hbm.at[pl.ds(ebase_of(j + 2), CD)], idx_b[b],
                         si_b[b])
        if not first:
            pltpu.make_async_copy(srows_b[b], acc_sh.at[dst_b[b]],
                                  ss_b[b]).wait()
        pltpu.async_copy(dst_hbm.at[pl.ds(ebase_of(j), CD)], dst_b[b],
                         sd_b[b])
        wait_h(b)
        if not first:
            pltpu.make_async_copy(a_b[b], a_hbm.at[pl.ds(0, CD)],
                                  sao_b[b]).wait()
        rows_v = rows_b[b]
        srows_v = srows_b[b]
        h_v = h_b[b]
        a_v = a_b[b]

        def group(g, carry2):
            avec = jnp.zeros((16,), jnp.float32)
            for l in range(4):
                hl = h_v[pl.ds(l * CD + 16 * g, 16)]
                gl = jnp.where(hl >= 0, hl, 0.01 * hl)
                avec = avec + jnp.exp(gl) * zs[l]
            a_v[pl.ds(16 * g, 16)] = avec
            svec = avec * scale
            for u in range(16):
                i = g * 16 + u
                sc = svec[u]
                for k in range(8):
                    srows_v[i, pl.ds(16 * k, 16)] = (
                        rows_v[i, pl.ds(16 * k, 16)] * sc)
            return carry2

        lax.fori_loop(0, CD // 16, group, 0, unroll=False)
        pltpu.make_async_copy(dst_hbm.at[pl.ds(0, CD)], dst_b[b],
                              sd_b[b]).wait()
        pltpu.async_copy(srows_v, acc_sh.at[dst_b[b]], ss_b[b], add=True)
        pltpu.async_copy(a_v, a_hbm.at[pl.ds(ebase_of(j), CD)], sao_b[b])
        pltpu.make_async_copy(src_hbm.at[pl.ds(0, CD)], idx_b[b],
                              si_b[b]).wait()
        add_off(idx_b[b])
        stage_h(j + 2, b)
        pltpu.async_copy(t2_hbm.at[idx_b[b]], rows_b[b], sg_b[b])

    for b in range(2):
        pltpu.sync_copy(src_hbm.at[pl.ds(ebase_of(b), CD)], idx_b[b])
        add_off(idx_b[b])
        stage_h(b, b)
        pltpu.async_copy(t2_hbm.at[idx_b[b]], rows_b[b], sg_b[b])
    process(0, 0, True)
    process(1, 1, True)

    def pair(g, carry):
        process(2 * g, 0, False)
        process(2 * g + 1, 1, False)
        return carry

    lax.fori_loop(1, n_loop // 2, pair, 0, unroll=False)
    for b in range(2):
        pltpu.make_async_copy(srows_b[b], acc_sh.at[dst_b[b]],
                              ss_b[b]).wait()
        pltpu.make_async_copy(t2_hbm.at[idx_b[b]], rows_b[b], sg_b[b]).wait()
        pltpu.make_async_copy(a_b[b], a_hbm.at[pl.ds(0, CD)],
                              sao_b[b]).wait()
        wait_h(b)

    plsc.subcore_barrier()
    for k in range(out_ch + 1):
        rb = rowbase(k)
        pltpu.sync_copy(acc_sh.at[pl.ds(rb, 16)],
                        out_hbm.at[c, pl.ds(rb, 16)])


def kernel(node_features, edge_features, W_node_w, W_node_b, Wp, bp, We, be,
           edge_index):
    n, d_node = node_features.shape
    e, d_edge = edge_features.shape
    l_att, d_trans, att = Wp.shape
    lat = l_att * att

    src = edge_index[0]
    dst = edge_index[1]
    w_cat = jnp.transpose(Wp, (1, 0, 2)).reshape(d_trans, lat)
    b_cat = bp.reshape(1, lat)

    nb = 400 if n % 400 == 0 else n
    grid_n = n // nb
    t2, p_packed = pl.pallas_call(
        _dense_nodes_body,
        grid=(grid_n,),
        in_specs=[
            pl.BlockSpec((nb, d_node), lambda i: (i, 0)),
            pl.BlockSpec((d_node, d_trans), lambda i: (0, 0)),
            pl.BlockSpec((1, d_trans), lambda i: (0, 0)),
            pl.BlockSpec((d_trans, lat), lambda i: (0, 0)),
            pl.BlockSpec((1, lat), lambda i: (0, 0)),
        ],
        out_specs=[
            pl.BlockSpec((2, nb, d_trans // 2), lambda i: (0, i, 0)),
            pl.BlockSpec((nb, lat), lambda i: (i, 0)),
        ],
        out_shape=[
            jax.ShapeDtypeStruct((2, n, d_trans // 2), jnp.float32),
            jax.ShapeDtypeStruct((n, lat), jnp.float32),
        ],
    )(node_features, W_node_w, W_node_b.reshape(1, d_trans), w_cat, b_cat)

    eb = 8000 if e % 8000 == 0 else e
    grid_e = e // eb
    r_edge = pl.pallas_call(
        _dense_edges_body,
        grid=(grid_e,),
        in_specs=[
            pl.BlockSpec((eb, d_edge), lambda i: (i, 0)),
            pl.BlockSpec((d_edge, att), lambda i: (0, 0)),
            pl.BlockSpec((1, att), lambda i: (0, 0)),
        ],
        out_specs=pl.BlockSpec((eb, att), lambda i: (i, 0)),
        out_shape=jax.ShapeDtypeStruct((e, att), jnp.float32),
    )(edge_features, We, be.reshape(1, att))

    t2_flat = t2.reshape(2 * n, d_trans // 2)

    n_ch_b = e // CB
    loop_n_b = n_ch_b // NUM_W + (1 if n_ch_b % NUM_W else 0)
    mesh = plsc.VectorSubcoreMesh(core_axis_name="c", subcore_axis_name="s")
    h_lin, partials = pl.kernel(
        functools.partial(_logits_body, n_ch_b, loop_n_b),
        out_type=(jax.ShapeDtypeStruct((l_att * e,), jnp.float32),
                  jax.ShapeDtypeStruct((NUM_W * 16,), jnp.float32)),
        mesh=mesh,
        compiler_params=pltpu.CompilerParams(needs_layout_passes=False),
        scratch_types=[
            pltpu.VMEM((loop_n_b * CB,), jnp.int32),
            pltpu.VMEM((CB, lat), jnp.float32),
            pltpu.VMEM((CB, lat), jnp.float32),
            pltpu.VMEM((CB, att), jnp.float32),
            pltpu.VMEM((CB, att), jnp.float32),
            pltpu.VMEM((l_att * CB,), jnp.float32),
            pltpu.VMEM((l_att * CB,), jnp.float32),
            pltpu.VMEM((l_att * 16,), jnp.float32),
            pltpu.VMEM((16,), jnp.float32),
            pltpu.SemaphoreType.DMA,
            pltpu.SemaphoreType.DMA,
            pltpu.SemaphoreType.DMA,
            pltpu.SemaphoreType.DMA,
        ],
    )(p_packed, r_edge, src)

    n_ch_sc = e // CD
    n_loop_d = (n_ch_sc + NUM_TEC - 1) // NUM_TEC
    n_loop_d += n_loop_d % 2
    out_ch = n // (NUM_TEC * 16)
    half = d_trans // 2
    out2, a_out = pl.kernel(
        functools.partial(_scatter_body, n, n_ch_sc, n_loop_d, out_ch),
        out_type=(jax.ShapeDtypeStruct((2, n, half), jnp.float32),
                  jax.ShapeDtypeStruct((e,), jnp.float32)),
        mesh=mesh,
        compiler_params=pltpu.CompilerParams(needs_layout_passes=False),
        scratch_types=[
            pltpu.VMEM((CD,), jnp.int32),
            pltpu.VMEM((CD,), jnp.int32),
            pltpu.VMEM((CD,), jnp.int32),
            pltpu.VMEM((CD,), jnp.int32),
            pltpu.VMEM((l_att * CD,), jnp.float32),
            pltpu.VMEM((l_att * CD,), jnp.float32),
            pltpu.VMEM((CD,), jnp.float32),
            pltpu.VMEM((CD,), jnp.float32),
            pltpu.VMEM((NUM_W * 16,), jnp.float32),
            pltpu.VMEM((CD, half), jnp.float32),
            pltpu.VMEM((CD, half), jnp.float32),
            pltpu.VMEM((CD, half), jnp.float32),
            pltpu.VMEM((CD, half), jnp.float32),
            pltpu.VMEM((16, half), jnp.float32),
            pltpu.VMEM_SHARED((n, half), jnp.float32),
        ] + [pltpu.SemaphoreType.DMA] * 13,
    )(t2_flat, h_lin, partials, src, dst)

    h_agg = jnp.concatenate([out2[0], out2[1]], axis=1)
    return (h_agg, a_out.reshape(e, 1))


# speedup vs baseline: 1.0109x; 1.0109x over previous
import functools

import jax
import jax.numpy as jnp
from jax import lax
from jax.experimental import pallas as pl
from jax.experimental.pallas import tpu as pltpu
from jax.experimental.pallas import tpu_sc as plsc

NUM_SC = 2
NUM_TEC = 16
NUM_W = NUM_SC * NUM_TEC

CB = 128
CD = 64


def _dense_nodes_body(x_ref, wn_ref, bn_ref, wc_ref, bc_ref, t2_ref, p_ref):
    t = jnp.dot(x_ref[...], wn_ref[...], preferred_element_type=jnp.float32)
    t = t + bn_ref[...]
    p = jnp.dot(t, wc_ref[...], preferred_element_type=jnp.float32)
    p_ref[...] = p + bc_ref[...]
    half = t.shape[1] // 2
    t2_ref[0] = t[:, :half]
    t2_ref[1] = t[:, half:]


def _dense_edges_body(ef_ref, we_ref, be_ref, r_ref):
    r = jnp.dot(ef_ref[...], we_ref[...], preferred_element_type=jnp.float32)
    r_ref[...] = r + be_ref[...]


def _logits_body(n_ch, loop_n, p_hbm, r_hbm, src_hbm, h_hbm, part_hbm,
                 idx_all, rows0, rows1, r0, r1, h0, h1, psum_v, pack_v,
                 sg0, sg1, sh0, sh1):
    c = lax.axis_index("c")
    s = lax.axis_index("s")
    wid = s * NUM_SC + c
    lanes = lax.iota(jnp.int32, 16)

    base_row = jnp.minimum(wid * loop_n, n_ch - loop_n)
    base_flat = pl.multiple_of(base_row * CB, 8)
    pltpu.sync_copy(src_hbm.at[pl.ds(base_flat, loop_n * CB)], idx_all)
    for l in range(4):
        psum_v[pl.ds(16 * l, 16)] = jnp.zeros((16,), jnp.float32)

    rows_b = (rows0, rows1)
    r_b = (r0, r1)
    h_b = (h0, h1)
    sg_b = (sg0, sg1)
    sh_b = (sh0, sh1)

    RROW = CB // 8

    def start(j, b):
        jj = jnp.minimum(j, loop_n - 1)
        rbase = pl.multiple_of((base_row + jj) * RROW, 8)
        pltpu.async_copy(p_hbm.at[idx_all.at[pl.ds(jj * CB, CB)]],
                         rows_b[b], sg_b[b])
        pltpu.async_copy(r_hbm.at[pl.ds(rbase, RROW)], r_b[b], sg_b[b])

    def process(j, b, first):
        jj = jnp.minimum(j, loop_n - 1)
        canon = jnp.where(base_row + jj >= wid * loop_n, 1.0, 0.0)
        pltpu.make_async_copy(p_hbm.at[idx_all.at[pl.ds(jj * CB, CB)]],
                              rows_b[b], sg_b[b]).wait()
        pltpu.make_async_copy(r_hbm.at[pl.ds(0, RROW)], r_b[b],
                              sg_b[b]).wait()
        if not first:
            pltpu.make_async_copy(h_b[b], h_hbm.at[pl.ds(0, 4 * CB)],
                                  sh_b[b]).wait()
        rows_v = rows_b[b]
        r_v = r_b[b]
        h_v = h_b[b]

        def group(g, carry2):
            hvecs = [jnp.zeros((16,), jnp.float32) for _ in range(4)]
            for u in range(16):
                i = g * 16 + u
                rrow = 2 * g + (u // 8)
                rcol = (u % 8) * 64
                rsegs = [r_v[rrow, pl.ds(rcol + 16 * k, 16)]
                         for k in range(4)]
                msk = lanes == u
                for l in range(4):
                    acc = rows_v[i, pl.ds(64 * l, 16)] * rsegs[0]
                    acc = acc + rows_v[i, pl.ds(64 * l + 16, 16)] * rsegs[1]
                    acc = acc + rows_v[i, pl.ds(64 * l + 32, 16)] * rsegs[2]
                    acc = acc + rows_v[i, pl.ds(64 * l + 48, 16)] * rsegs[3]
                    hvecs[l] = jnp.where(msk, plsc.cumsum(acc)[15], hvecs[l])
            for l in range(4):
                hv = hvecs[l]
                h_v[pl.ds(128 * l + 16 * g, 16)] = hv
                glv = jnp.where(hv >= 0, hv, 0.01 * hv)
                psum_v[pl.ds(16 * l, 16)] = (
                    psum_v[pl.ds(16 * l, 16)] + jnp.exp(glv) * canon)
            return carry2

        lax.fori_loop(0, CB // 16, group, 0, unroll=False)
        pltpu.async_copy(
            h_v, h_hbm.at[pl.ds((base_row + jj) * (4 * CB), 4 * CB)],
            sh_b[b])
        start(j + 2, b)

    start(0, 0)
    start(1, 1)
    process(0, 0, True)
    process(1, 1, True)

    def pair(g, carry):
        process(2 * g, 0, False)
        process(2 * g + 1, 1, False)
        return carry

    lax.fori_loop(1, (loop_n + 1) // 2, pair, 0, unroll=False)
    if loop_n % 2:
        process(loop_n - 1, 0, False)
    for b in range(2):
        pltpu.make_async_copy(p_hbm.at[idx_all.at[pl.ds(0, CB)]], rows_b[b],
                              sg_b[b]).wait()
        pltpu.make_async_copy(r_hbm.at[pl.ds(0, RROW)], r_b[b],
                              sg_b[b]).wait()
        pltpu.make_async_copy(h_b[b], h_hbm.at[pl.ds(0, 4 * CB)],
                              sh_b[b]).wait()

    pack = jnp.zeros((16,), jnp.float32)
    for l in range(4):
        tot = plsc.cumsum(psum_v[pl.ds(16 * l, 16)])[15]
        pack = jnp.where(lanes == l, tot, pack)
    pack_v[...] = pack
    pltpu.sync_copy(pack_v, part_hbm.at[pl.ds(wid * 16, 16)])


def _scatter_body(n_nodes, n_ch, n_loop, out_ch, t2_hbm, h_hbm, part_hbm,
                  src_hbm, dst_hbm, out_hbm, a_hbm, idx0, idx1, dst0, dst1,
                  hb0, hb1, ab0, ab1, pbuf, rows0, rows1, srows0, srows1,
                  zbuf, acc_sh, sg0, sg1, ss0, ss1, si0, si1, sd0, sd1,
                  sa0, sa1, sao0, sao1, so):
    c = lax.axis_index("c")
    s = lax.axis_index("s")
    lanes = lax.iota(jnp.int32, 16)
    off = c * n_nodes

    pltpu.sync_copy(part_hbm, pbuf)
    zacc = jnp.zeros((16,), jnp.float32)
    for w in range(NUM_W):
        zacc = zacc + pbuf[pl.ds(16 * w, 16)]
    zinv = jnp.where(lanes < 4, 0.25 / zacc, 0.0)
    zs = [zinv[l] for l in range(4)]

    def zfill(r, carry):
        for k in range(8):
            zbuf[r, pl.ds(16 * k, 16)] = jnp.zeros((16,), jnp.float32)
        return carry

    lax.fori_loop(0, 16, zfill, 0, unroll=False)

    def rowbase(k):
        rb = jnp.minimum(s * (out_ch * 16) + 16 * k, n_nodes - 16)
        return pl.multiple_of(rb, 8)

    for k in range(out_ch + 1):
        pltpu.sync_copy(zbuf, acc_sh.at[pl.ds(rowbase(k), 16)])
    plsc.subcore_barrier()

    idx_b = (idx0, idx1)
    dst_b = (dst0, dst1)
    h_b = (hb0, hb1)
    a_b = (ab0, ab1)
    rows_b = (rows0, rows1)
    srows_b = (srows0, srows1)
    sg_b = (sg0, sg1)
    ss_b = (ss0, ss1)
    si_b = (si0, si1)
    sd_b = (sd0, sd1)
    sh_b = (sa0, sa1)
    sao_b = (sao0, sao1)

    def cid_of(j):
        return jnp.minimum(j * NUM_TEC + s, n_ch - 1)

    def ebase_of(j):
        return pl.multiple_of(cid_of(j) * CD, 8)

    def stage_h(j, b):
        cid = cid_of(j)
        bbase = (cid // 2) * (4 * CB) + (cid % 2) * CD
        for l in range(4):
            pltpu.async_copy(h_hbm.at[pl.ds(bbase + l * CB, CD)],
                             h_b[b].at[pl.ds(l * CD, CD)], sh_b[b])

    def wait_h(b):
        for l in range(4):
            pltpu.make_async_copy(h_hbm.at[pl.ds(0, CD)],
                                  h_b[b].at[pl.ds(l * CD, CD)],
                                  sh_b[b]).wait()

    def add_off(buf):
        for k in range(CD // 16):
            buf[pl.ds(16 * k, 16)] = buf[pl.ds(16 * k, 16)] + off

    def process(j, b, first):
        scale = jnp.where(j * NUM_TEC + s < n_ch, 1.0, 0.0)
        pltpu.make_async_copy(t2_hbm.at[idx_b[b]], rows_b[b], sg_b[b]).wait()
        pltpu.async_copy(src_hbm.at[pl.ds(ebase_of(j + 2), CD)], idx_b[b],
                         si_b[b])
        if not first:
            pltpu.make_async_copy(srows_b[b], acc_sh.at[dst_b[b]],
                                  ss_b[b]).wait()
        pltpu.async_copy(dst_hbm.at[pl.ds(ebase_of(j), CD)], dst_b[b],
                         sd_b[b])
        wait_h(b)
        if not first:
            pltpu.make_async_copy(a_b[b], a_hbm.at[pl.ds(0, CD)],
                                  sao_b[b]).wait()
        rows_v = rows_b[b]
        srows_v = srows_b[b]
        h_v = h_b[b]
        a_v = a_b[b]

        def group(g, carry2):
            avec = jnp.zeros((16,), jnp.float32)
            for l in range(4):
                hl = h_v[pl.ds(l * CD + 16 * g, 16)]
                gl = jnp.where(hl >= 0, hl, 0.01 * hl)
                avec = avec + jnp.exp(gl) * zs[l]
            a_v[pl.ds(16 * g, 16)] = avec
            svec = avec * scale
            for u in range(16):
                i = g * 16 + u
                sc = svec[u]
                for k in range(8):
                    srows_v[i, pl.ds(16 * k, 16)] = (
                        rows_v[i, pl.ds(16 * k, 16)] * sc)
            return carry2

        lax.fori_loop(0, CD // 16, group, 0, unroll=False)
        pltpu.make_async_copy(dst_hbm.at[pl.ds(0, CD)], dst_b[b],
                              sd_b[b]).wait()
        pltpu.async_copy(srows_v, acc_sh.at[dst_b[b]], ss_b[b], add=True)
        pltpu.async_copy(a_v, a_hbm.at[pl.ds(ebase_of(j), CD)], sao_b[b])
        pltpu.make_async_copy(src_hbm.at[pl.ds(0, CD)], idx_b[b],
                              si_b[b]).wait()
        add_off(idx_b[b])
        stage_h(j + 2, b)
        pltpu.async_copy(t2_hbm.at[idx_b[b]], rows_b[b], sg_b[b])

    for b in range(2):
        pltpu.sync_copy(src_hbm.at[pl.ds(ebase_of(b), CD)], idx_b[b])
        add_off(idx_b[b])
        stage_h(b, b)
        pltpu.async_copy(t2_hbm.at[idx_b[b]], rows_b[b], sg_b[b])
    process(0, 0, True)
    process(1, 1, True)

    def pair(g, carry):
        process(2 * g, 0, False)
        process(2 * g + 1, 1, False)
        return carry

    lax.fori_loop(1, n_loop // 2, pair, 0, unroll=False)
    for b in range(2):
        pltpu.make_async_copy(srows_b[b], acc_sh.at[dst_b[b]],
                              ss_b[b]).wait()
        pltpu.make_async_copy(t2_hbm.at[idx_b[b]], rows_b[b], sg_b[b]).wait()
        pltpu.make_async_copy(a_b[b], a_hbm.at[pl.ds(0, CD)],
                              sao_b[b]).wait()
        wait_h(b)

    plsc.subcore_barrier()
    for k in range(out_ch + 1):
        rb = rowbase(k)
        pltpu.sync_copy(acc_sh.at[pl.ds(rb, 16)],
                        out_hbm.at[c, pl.ds(rb, 16)])


def kernel(node_features, edge_features, W_node_w, W_node_b, Wp, bp, We, be,
           edge_index):
    n, d_node = node_features.shape
    e, d_edge = edge_features.shape
    l_att, d_trans, att = Wp.shape
    lat = l_att * att

    src = edge_index[0]
    dst = edge_index[1]
    w_cat = jnp.transpose(Wp, (1, 0, 2)).reshape(d_trans, lat)
    b_cat = bp.reshape(1, lat)

    nb = 200 if n % 200 == 0 else n
    grid_n = n // nb
    t2, p_packed = pl.pallas_call(
        _dense_nodes_body,
        grid=(grid_n,),
        in_specs=[
            pl.BlockSpec((nb, d_node), lambda i: (i, 0)),
            pl.BlockSpec((d_node, d_trans), lambda i: (0, 0)),
            pl.BlockSpec((1, d_trans), lambda i: (0, 0)),
            pl.BlockSpec((d_trans, lat), lambda i: (0, 0)),
            pl.BlockSpec((1, lat), lambda i: (0, 0)),
        ],
        out_specs=[
            pl.BlockSpec((2, nb, d_trans // 2), lambda i: (0, i, 0)),
            pl.BlockSpec((nb, lat), lambda i: (i, 0)),
        ],
        out_shape=[
            jax.ShapeDtypeStruct((2, n, d_trans // 2), jnp.float32),
            jax.ShapeDtypeStruct((n, lat), jnp.float32),
        ],
    )(node_features, W_node_w, W_node_b.reshape(1, d_trans), w_cat, b_cat)

    ef2 = edge_features.reshape(e // 8, 8 * d_edge)
    w_big = jnp.kron(jnp.eye(8, dtype=jnp.float32), We)
    b_big = jnp.tile(be, 8).reshape(1, 8 * att)
    e8 = e // 8
    eb = 1000 if e8 % 1000 == 0 else e8
    grid_e = e8 // eb
    r_edge = pl.pallas_call(
        _dense_edges_body,
        grid=(grid_e,),
        in_specs=[
            pl.BlockSpec((eb, 8 * d_edge), lambda i: (i, 0)),
            pl.BlockSpec((8 * d_edge, 8 * att), lambda i: (0, 0)),
            pl.BlockSpec((1, 8 * att), lambda i: (0, 0)),
        ],
        out_specs=pl.BlockSpec((eb, 8 * att), lambda i: (i, 0)),
        out_shape=jax.ShapeDtypeStruct((e8, 8 * att), jnp.float32),
    )(ef2, w_big, b_big)

    t2_flat = t2.reshape(2 * n, d_trans // 2)

    n_ch_b = e // CB
    loop_n_b = n_ch_b // NUM_W + (1 if n_ch_b % NUM_W else 0)
    mesh = plsc.VectorSubcoreMesh(core_axis_name="c", subcore_axis_name="s")
    h_lin, partials = pl.kernel(
        functools.partial(_logits_body, n_ch_b, loop_n_b),
        out_type=(jax.ShapeDtypeStruct((l_att * e,), jnp.float32),
                  jax.ShapeDtypeStruct((NUM_W * 16,), jnp.float32)),
        mesh=mesh,
        compiler_params=pltpu.CompilerParams(needs_layout_passes=False),
        scratch_types=[
            pltpu.VMEM((loop_n_b * CB,), jnp.int32),
            pltpu.VMEM((CB, lat), jnp.float32),
            pltpu.VMEM((CB, lat), jnp.float32),
            pltpu.VMEM((CB // 8, 8 * att), jnp.float32),
            pltpu.VMEM((CB // 8, 8 * att), jnp.float32),
            pltpu.VMEM((l_att * CB,), jnp.float32),
            pltpu.VMEM((l_att * CB,), jnp.float32),
            pltpu.VMEM((l_att * 16,), jnp.float32),
            pltpu.VMEM((16,), jnp.float32),
            pltpu.SemaphoreType.DMA,
            pltpu.SemaphoreType.DMA,
            pltpu.SemaphoreType.DMA,
            pltpu.SemaphoreType.DMA,
        ],
    )(p_packed, r_edge, src)

    n_ch_sc = e // CD
    n_loop_d = (n_ch_sc + NUM_TEC - 1) // NUM_TEC
    n_loop_d += n_loop_d % 2
    out_ch = n // (NUM_TEC * 16)
    half = d_trans // 2
    out2, a_out = pl.kernel(
        functools.partial(_scatter_body, n, n_ch_sc, n_loop_d, out_ch),
        out_type=(jax.ShapeDtypeStruct((2, n, half), jnp.float32),
                  jax.ShapeDtypeStruct((e,), jnp.float32)),
        mesh=mesh,
        compiler_params=pltpu.CompilerParams(needs_layout_passes=False),
        scratch_types=[
            pltpu.VMEM((CD,), jnp.int32),
            pltpu.VMEM((CD,), jnp.int32),
            pltpu.VMEM((CD,), jnp.int32),
            pltpu.VMEM((CD,), jnp.int32),
            pltpu.VMEM((l_att * CD,), jnp.float32),
            pltpu.VMEM((l_att * CD,), jnp.float32),
            pltpu.VMEM((CD,), jnp.float32),
            pltpu.VMEM((CD,), jnp.float32),
            pltpu.VMEM((NUM_W * 16,), jnp.float32),
            pltpu.VMEM((CD, half), jnp.float32),
            pltpu.VMEM((CD, half), jnp.float32),
            pltpu.VMEM((CD, half), jnp.float32),
            pltpu.VMEM((CD, half), jnp.float32),
            pltpu.VMEM((16, half), jnp.float32),
            pltpu.VMEM_SHARED((n, half), jnp.float32),
        ] + [pltpu.SemaphoreType.DMA] * 13,
    )(t2_flat, h_lin, partials, src, dst)

    h_agg = jnp.concatenate([out2[0], out2[1]], axis=1)
    return (h_agg, a_out.reshape(e, 1))


# speedup vs baseline: 1.0944x; 1.0826x over previous
import functools

import jax
import jax.numpy as jnp
from jax import lax
from jax.experimental import pallas as pl
from jax.experimental.pallas import tpu as pltpu
from jax.experimental.pallas import tpu_sc as plsc

NUM_SC = 2
NUM_TEC = 16
NUM_W = NUM_SC * NUM_TEC

CB = 128
CD = 64


def _dense_nodes_body(x_ref, wn_ref, bn_ref, wc_ref, bc_ref, t2_ref, p_ref):
    t = jnp.dot(x_ref[...], wn_ref[...], preferred_element_type=jnp.float32)
    t = t + bn_ref[...]
    p = jnp.dot(t, wc_ref[...], preferred_element_type=jnp.float32)
    p_ref[...] = p + bc_ref[...]
    half = t.shape[1] // 2
    t2_ref[0] = t[:, :half]
    t2_ref[1] = t[:, half:]


def _dense_edges_body(ef_ref, we_ref, be_ref, r_ref):
    r = jnp.dot(ef_ref[...], we_ref[...], preferred_element_type=jnp.float32)
    r_ref[...] = r + be_ref[...]


def _logits_body(n_ch, loop_n, p_hbm, r_hbm, src_hbm, h_hbm, part_hbm,
                 idx_all, rows0, rows1, r0, r1, h0, h1, psum_v, pack_v,
                 sg0, sg1, sh0, sh1):
    c = lax.axis_index("c")
    s = lax.axis_index("s")
    wid = s * NUM_SC + c
    lanes = lax.iota(jnp.int32, 16)

    base_row = jnp.minimum(wid * loop_n, n_ch - loop_n)
    base_flat = pl.multiple_of(base_row * CB, 8)
    pltpu.sync_copy(src_hbm.at[pl.ds(base_flat, loop_n * CB)], idx_all)
    for l in range(4):
        psum_v[pl.ds(16 * l, 16)] = jnp.zeros((16,), jnp.float32)

    rows_b = (rows0, rows1)
    r_b = (r0, r1)
    h_b = (h0, h1)
    sg_b = (sg0, sg1)
    sh_b = (sh0, sh1)

    RROW = CB // 8

    def start(j, b):
        jj = jnp.minimum(j, loop_n - 1)
        rbase = pl.multiple_of((base_row + jj) * RROW, 8)
        pltpu.async_copy(p_hbm.at[idx_all.at[pl.ds(jj * CB, CB)]],
                         rows_b[b], sg_b[b])
        pltpu.async_copy(r_hbm.at[pl.ds(rbase, RROW)], r_b[b], sg_b[b])

    def process(j, b, first):
        jj = jnp.minimum(j, loop_n - 1)
        canon = jnp.where(base_row + jj >= wid * loop_n, 1.0, 0.0)
        pltpu.make_async_copy(p_hbm.at[idx_all.at[pl.ds(jj * CB, CB)]],
                              rows_b[b], sg_b[b]).wait()
        pltpu.make_async_copy(r_hbm.at[pl.ds(0, RROW)], r_b[b],
                              sg_b[b]).wait()
        if not first:
            pltpu.make_async_copy(h_b[b], h_hbm.at[pl.ds(0, 4 * CB)],
                                  sh_b[b]).wait()
        rows_v = rows_b[b]
        r_v = r_b[b]
        h_v = h_b[b]

        def group(g, carry2):
            hvecs = [jnp.zeros((16,), jnp.float32) for _ in range(4)]
            for u in range(16):
                i = g * 16 + u
                rrow = 2 * g + (u // 8)
                rcol = (u % 8) * 64
                rsegs = [r_v[rrow, pl.ds(rcol + 16 * k, 16)]
                         for k in range(4)]
                msk = lanes == u
                for l in range(4):
                    acc = rows_v[i, pl.ds(64 * l, 16)] * rsegs[0]
                    acc = acc + rows_v[i, pl.ds(64 * l + 16, 16)] * rsegs[1]
                    acc = acc + rows_v[i, pl.ds(64 * l + 32, 16)] * rsegs[2]
                    acc = acc + rows_v[i, pl.ds(64 * l + 48, 16)] * rsegs[3]
                    hvecs[l] = jnp.where(msk, plsc.cumsum(acc)[15], hvecs[l])
            for l in range(4):
                hv = hvecs[l]
                h_v[pl.ds(128 * l + 16 * g, 16)] = hv
                glv = jnp.where(hv >= 0, hv, 0.01 * hv)
                psum_v[pl.ds(16 * l, 16)] = (
                    psum_v[pl.ds(16 * l, 16)] + jnp.exp(glv) * canon)
            return carry2

        lax.fori_loop(0, CB // 16, group, 0, unroll=False)
        pltpu.async_copy(
            h_v, h_hbm.at[pl.ds((base_row + jj) * (4 * CB), 4 * CB)],
            sh_b[b])
        start(j + 2, b)

    start(0, 0)
    start(1, 1)
    process(0, 0, True)
    process(1, 1, True)

    def pair(g, carry):
        process(2 * g, 0, False)
        process(2 * g + 1, 1, False)
        return carry

    lax.fori_loop(1, (loop_n + 1) // 2, pair, 0, unroll=False)
    if loop_n % 2:
        process(loop_n - 1, 0, False)
    for b in range(2):
        pltpu.make_async_copy(p_hbm.at[idx_all.at[pl.ds(0, CB)]], rows_b[b],
                              sg_b[b]).wait()
        pltpu.make_async_copy(r_hbm.at[pl.ds(0, RROW)], r_b[b],
                              sg_b[b]).wait()
        pltpu.make_async_copy(h_b[b], h_hbm.at[pl.ds(0, 4 * CB)],
                              sh_b[b]).wait()

    pack = jnp.zeros((16,), jnp.float32)
    for l in range(4):
        tot = plsc.cumsum(psum_v[pl.ds(16 * l, 16)])[15]
        pack = jnp.where(lanes == l, tot, pack)
    pack_v[...] = pack
    pltpu.sync_copy(pack_v, part_hbm.at[pl.ds(wid * 16, 16)])


def _scatter_body(n_nodes, n_ch, n_loop, out_ch, t2_hbm, h_hbm, part_hbm,
                  src_hbm, dst_hbm, out_hbm, a_hbm, idx0, idx1, dst0, dst1,
                  hb0, hb1, ab0, ab1, pbuf, rows0, rows1, srows0, srows1,
                  zbuf, acc_sh, sg0, sg1, ss0, ss1, si0, si1, sd0, sd1,
                  sa0, sa1, sao0, sao1, so):
    c = lax.axis_index("c")
    s = lax.axis_index("s")
    lanes = lax.iota(jnp.int32, 16)
    off = c * n_nodes

    pltpu.sync_copy(part_hbm, pbuf)
    zacc = jnp.zeros((16,), jnp.float32)
    for w in range(NUM_W):
        zacc = zacc + pbuf[pl.ds(16 * w, 16)]
    zinv = jnp.where(lanes < 4, 0.25 / zacc, 0.0)
    zs = [zinv[l] for l in range(4)]

    def zfill(r, carry):
        for k in range(8):
            zbuf[r, pl.ds(16 * k, 16)] = jnp.zeros((16,), jnp.float32)
        return carry

    lax.fori_loop(0, 16, zfill, 0, unroll=False)

    def rowbase(k):
        rb = jnp.minimum(s * (out_ch * 16) + 16 * k, n_nodes - 16)
        return pl.multiple_of(rb, 8)

    for k in range(out_ch + 1):
        pltpu.async_copy(zbuf, acc_sh.at[pl.ds(rowbase(k), 16)], so)
        if k % 4 == 3 or k == out_ch:
            for _ in range(4 if k % 4 == 3 else k % 4 + 1):
                pltpu.make_async_copy(zbuf, acc_sh.at[pl.ds(0, 16)],
                                      so).wait()
    plsc.subcore_barrier()

    idx_b = (idx0, idx1)
    dst_b = (dst0, dst1)
    h_b = (hb0, hb1)
    a_b = (ab0, ab1)
    rows_b = (rows0, rows1)
    srows_b = (srows0, srows1)
    sg_b = (sg0, sg1)
    ss_b = (ss0, ss1)
    si_b = (si0, si1)
    sd_b = (sd0, sd1)
    sh_b = (sa0, sa1)
    sao_b = (sao0, sao1)

    def cid_of(j):
        return jnp.minimum(j * NUM_TEC + s, n_ch - 1)

    def ebase_of(j):
        return pl.multiple_of(cid_of(j) * CD, 8)

    def stage_h(j, b):
        cid = cid_of(j)
        bbase = (cid // 2) * (4 * CB) + (cid % 2) * CD
        for l in range(4):
            pltpu.async_copy(h_hbm.at[pl.ds(bbase + l * CB, CD)],
                             h_b[b].at[pl.ds(l * CD, CD)], sh_b[b])

    def wait_h(b):
        for l in range(4):
            pltpu.make_async_copy(h_hbm.at[pl.ds(0, CD)],
                                  h_b[b].at[pl.ds(l * CD, CD)],
                                  sh_b[b]).wait()

    def add_off(buf):
        for k in range(CD // 16):
            buf[pl.ds(16 * k, 16)] = buf[pl.ds(16 * k, 16)] + off

    def process(j, b, first):
        scale = jnp.where(j * NUM_TEC + s < n_ch, 1.0, 0.0)
        pltpu.make_async_copy(t2_hbm.at[idx_b[b]], rows_b[b], sg_b[b]).wait()
        pltpu.async_copy(src_hbm.at[pl.ds(ebase_of(j + 2), CD)], idx_b[b],
                         si_b[b])
        if not first:
            pltpu.make_async_copy(srows_b[b], acc_sh.at[dst_b[b]],
                                  ss_b[b]).wait()
        pltpu.async_copy(dst_hbm.at[pl.ds(ebase_of(j), CD)], dst_b[b],
                         sd_b[b])
        wait_h(b)
        if not first:
            pltpu.make_async_copy(a_b[b], a_hbm.at[pl.ds(0, CD)],
                                  sao_b[b]).wait()
        rows_v = rows_b[b]
        srows_v = srows_b[b]
        h_v = h_b[b]
        a_v = a_b[b]

        def group(g, carry2):
            avec = jnp.zeros((16,), jnp.float32)
            for l in range(4):
                hl = h_v[pl.ds(l * CD + 16 * g, 16)]
                gl = jnp.where(hl >= 0, hl, 0.01 * hl)
                avec = avec + jnp.exp(gl) * zs[l]
            a_v[pl.ds(16 * g, 16)] = avec
            svec = avec * scale
            for u in range(16):
                i = g * 16 + u
                sc = svec[u]
                for k in range(8):
                    srows_v[i, pl.ds(16 * k, 16)] = (
                        rows_v[i, pl.ds(16 * k, 16)] * sc)
            return carry2

        lax.fori_loop(0, CD // 16, group, 0, unroll=False)
        pltpu.make_async_copy(dst_hbm.at[pl.ds(0, CD)], dst_b[b],
                              sd_b[b]).wait()
        pltpu.async_copy(srows_v, acc_sh.at[dst_b[b]], ss_b[b], add=True)
        pltpu.async_copy(a_v, a_hbm.at[pl.ds(ebase_of(j), CD)], sao_b[b])
        pltpu.make_async_copy(src_hbm.at[pl.ds(0, CD)], idx_b[b],
                              si_b[b]).wait()
        add_off(idx_b[b])
        stage_h(j + 2, b)
        pltpu.async_copy(t2_hbm.at[idx_b[b]], rows_b[b], sg_b[b])

    for b in range(2):
        pltpu.sync_copy(src_hbm.at[pl.ds(ebase_of(b), CD)], idx_b[b])
        add_off(idx_b[b])
        stage_h(b, b)
        pltpu.async_copy(t2_hbm.at[idx_b[b]], rows_b[b], sg_b[b])
    process(0, 0, True)
    process(1, 1, True)

    def pair(g, carry):
        process(2 * g, 0, False)
        process(2 * g + 1, 1, False)
        return carry

    lax.fori_loop(1, n_loop // 2, pair, 0, unroll=False)
    for b in range(2):
        pltpu.make_async_copy(srows_b[b], acc_sh.at[dst_b[b]],
                              ss_b[b]).wait()
        pltpu.make_async_copy(t2_hbm.at[idx_b[b]], rows_b[b], sg_b[b]).wait()
        pltpu.make_async_copy(a_b[b], a_hbm.at[pl.ds(0, CD)],
                              sao_b[b]).wait()
        wait_h(b)

    plsc.subcore_barrier()
    for k in range(out_ch + 1):
        rb = rowbase(k)
        pltpu.async_copy(acc_sh.at[pl.ds(rb, 16)],
                         out_hbm.at[c, pl.ds(rb, 16)], so)
        if k % 4 == 3 or k == out_ch:
            for _ in range(4 if k % 4 == 3 else k % 4 + 1):
                pltpu.make_async_copy(acc_sh.at[pl.ds(0, 16)],
                                      out_hbm.at[c, pl.ds(0, 16)],
                                      so).wait()


def kernel(node_features, edge_features, W_node_w, W_node_b, Wp, bp, We, be,
           edge_index):
    n, d_node = node_features.shape
    e, d_edge = edge_features.shape
    l_att, d_trans, att = Wp.shape
    lat = l_att * att

    src = edge_index[0]
    dst = edge_index[1]
    w_cat = jnp.transpose(Wp, (1, 0, 2)).reshape(d_trans, lat)
    b_cat = bp.reshape(1, lat)

    nb = 400 if n % 400 == 0 else n
    grid_n = n // nb
    t2, p_packed = pl.pallas_call(
        _dense_nodes_body,
        grid=(grid_n,),
        in_specs=[
            pl.BlockSpec((nb, d_node), lambda i: (i, 0)),
            pl.BlockSpec((d_node, d_trans), lambda i: (0, 0)),
            pl.BlockSpec((1, d_trans), lambda i: (0, 0)),
            pl.BlockSpec((d_trans, lat), lambda i: (0, 0)),
            pl.BlockSpec((1, lat), lambda i: (0, 0)),
        ],
        out_specs=[
            pl.BlockSpec((2, nb, d_trans // 2), lambda i: (0, i, 0)),
            pl.BlockSpec((nb, lat), lambda i: (i, 0)),
        ],
        out_shape=[
            jax.ShapeDtypeStruct((2, n, d_trans // 2), jnp.float32),
            jax.ShapeDtypeStruct((n, lat), jnp.float32),
        ],
    )(node_features, W_node_w, W_node_b.reshape(1, d_trans), w_cat, b_cat)

    ef2 = edge_features.reshape(e // 8, 8 * d_edge)
    w_big = jnp.kron(jnp.eye(8, dtype=jnp.float32), We)
    b_big = jnp.tile(be, 8).reshape(1, 8 * att)
    e8 = e // 8
    eb = 1000 if e8 % 1000 == 0 else e8
    grid_e = e8 // eb
    r_edge = pl.pallas_call(
        _dense_edges_body,
        grid=(grid_e,),
        in_specs=[
            pl.BlockSpec((eb, 8 * d_edge), lambda i: (i, 0)),
            pl.BlockSpec((8 * d_edge, 8 * att), lambda i: (0, 0)),
            pl.BlockSpec((1, 8 * att), lambda i: (0, 0)),
        ],
        out_specs=pl.BlockSpec((eb, 8 * att), lambda i: (i, 0)),
        out_shape=jax.ShapeDtypeStruct((e8, 8 * att), jnp.float32),
    )(ef2, w_big, b_big)

    t2_flat = t2.reshape(2 * n, d_trans // 2)

    n_ch_b = e // CB
    loop_n_b = n_ch_b // NUM_W + (1 if n_ch_b % NUM_W else 0)
    mesh = plsc.VectorSubcoreMesh(core_axis_name="c", subcore_axis_name="s")
    h_lin, partials = pl.kernel(
        functools.partial(_logits_body, n_ch_b, loop_n_b),
        out_type=(jax.ShapeDtypeStruct((l_att * e,), jnp.float32),
                  jax.ShapeDtypeStruct((NUM_W * 16,), jnp.float32)),
        mesh=mesh,
        compiler_params=pltpu.CompilerParams(needs_layout_passes=False),
        scratch_types=[
            pltpu.VMEM((loop_n_b * CB,), jnp.int32),
            pltpu.VMEM((CB, lat), jnp.float32),
            pltpu.VMEM((CB, lat), jnp.float32),
            pltpu.VMEM((CB // 8, 8 * att), jnp.float32),
            pltpu.VMEM((CB // 8, 8 * att), jnp.float32),
            pltpu.VMEM((l_att * CB,), jnp.float32),
            pltpu.VMEM((l_att * CB,), jnp.float32),
            pltpu.VMEM((l_att * 16,), jnp.float32),
            pltpu.VMEM((16,), jnp.float32),
            pltpu.SemaphoreType.DMA,
            pltpu.SemaphoreType.DMA,
            pltpu.SemaphoreType.DMA,
            pltpu.SemaphoreType.DMA,
        ],
    )(p_packed, r_edge, src)

    n_ch_sc = e // CD
    n_loop_d = (n_ch_sc + NUM_TEC - 1) // NUM_TEC
    n_loop_d += n_loop_d % 2
    out_ch = n // (NUM_TEC * 16)
    half = d_trans // 2
    out2, a_out = pl.kernel(
        functools.partial(_scatter_body, n, n_ch_sc, n_loop_d, out_ch),
        out_type=(jax.ShapeDtypeStruct((2, n, half), jnp.float32),
                  jax.ShapeDtypeStruct((e,), jnp.float32)),
        mesh=mesh,
        compiler_params=pltpu.CompilerParams(needs_layout_passes=False),
        scratch_types=[
            pltpu.VMEM((CD,), jnp.int32),
            pltpu.VMEM((CD,), jnp.int32),
            pltpu.VMEM((CD,), jnp.int32),
            pltpu.VMEM((CD,), jnp.int32),
            pltpu.VMEM((l_att * CD,), jnp.float32),
            pltpu.VMEM((l_att * CD,), jnp.float32),
            pltpu.VMEM((CD,), jnp.float32),
            pltpu.VMEM((CD,), jnp.float32),
            pltpu.VMEM((NUM_W * 16,), jnp.float32),
            pltpu.VMEM((CD, half), jnp.float32),
            pltpu.VMEM((CD, half), jnp.float32),
            pltpu.VMEM((CD, half), jnp.float32),
            pltpu.VMEM((CD, half), jnp.float32),
            pltpu.VMEM((16, half), jnp.float32),
            pltpu.VMEM_SHARED((n, half), jnp.float32),
        ] + [pltpu.SemaphoreType.DMA] * 13,
    )(t2_flat, h_lin, partials, src, dst)

    h_agg = jnp.concatenate([out2[0], out2[1]], axis=1)
    return (h_agg, a_out.reshape(e, 1))
